# Initial kernel scaffold; baseline (speedup 1.0000x reference)
import jax
import jax.numpy as jnp
from jax.experimental import pallas as pl


def kernel(_x, _edge_index, _edge_weight, W1, u1, c1, b1, W2, u2, c2, b2, Wl, bl, Wg1, bg1, Wmu, bmu, Wlv, blv, Wd1, bd1, Wd2, bd2):
    raise NotImplementedError("write your pallas kernel here")


# speedup vs baseline: 5.2842x; 5.2842x over previous
import functools

import jax
import jax.numpy as jnp
from jax import lax
from jax.experimental import pallas as pl
from jax.experimental.pallas import tpu as pltpu
from jax.experimental.pallas import tpu_sc as plsc

N = 10000
E = 320000
F_IN, HID, LAT, OUT = 128, 256, 64, 128
HEADS = 2

CH = 80
TRS = 624
TRN = 640
NCH_FULL = (E // 16) // CH
NCH_HALF = (E // 32) // CH

f32 = jnp.float32


def _zero_rows(buf):
    def body(i, carry):
        for k in range(8):
            buf[i, pl.ds(k * 16, 16)] = jnp.zeros((16,), f32)
        return carry
    lax.fori_loop(0, CH, body, 0)


def _zero_vec(buf):
    for g in range(CH // 16):
        buf[pl.ds(g * 16, 16)] = jnp.zeros((16,), f32)


def _ones_vec(buf):
    for g in range(CH // 16):
        buf[pl.ds(g * 16, 16)] = jnp.ones((16,), f32)


def _row0(s):
    return pl.multiple_of(s * TRS, 8)


def _zero_acc(acc_sh, rows_v, s):
    r0 = _row0(s)
    for k in range(TRN // CH):
        pltpu.sync_copy(rows_v, acc_sh.at[pl.ds(r0 + k * CH, CH)])


def _flush_acc(acc_sh, out_ref, s, base=0):
    r0 = _row0(s)
    for k in range(TRN // CH):
        off = pl.multiple_of(base + r0 + k * CH, 8)
        pltpu.sync_copy(acc_sh.at[pl.ds(r0 + k * CH, CH)],
                        out_ref.at[pl.ds(off, CH)])


def _zero_sca(sca_sh, zvec):
    def body(i, carry):
        off = pl.multiple_of(i * CH, CH)
        pltpu.sync_copy(zvec, sca_sh.at[pl.ds(off, CH)])
        return carry
    lax.fori_loop(0, N // CH, body, 0)


def _gate(idx_v, dst_v, tA, tB, q_v):
    for g in range(CH // 16):
        si = idx_v[pl.ds(g * 16, 16)]
        di = dst_v[pl.ds(g * 16, 16)]
        a = plsc.load_gather(tA, [si])
        b = plsc.load_gather(tB, [di])
        q_v[pl.ds(g * 16, 16)] = 1.0 / (1.0 + jnp.exp(b - a))


def _norm(idx_v, dst_v, dt, w_v, q_v):
    for g in range(CH // 16):
        si = idx_v[pl.ds(g * 16, 16)]
        di = dst_v[pl.ds(g * 16, 16)]
        a = plsc.load_gather(dt, [si])
        b = plsc.load_gather(dt, [di])
        q_v[pl.ds(g * 16, 16)] = a * b * w_v[pl.ds(g * 16, 16)]


def _scale_rows(rows_v, q_v):
    def body(j, carry):
        qb = plsc.load_gather(q_v, [jnp.full((16,), j, jnp.int32)])
        for k in range(8):
            rows_v[j, pl.ds(k * 16, 16)] = rows_v[j, pl.ds(k * 16, 16)] * qb
        return carry
    lax.fori_loop(0, CH, body, 0)


def _offset_idx(idx_v, off):
    ov = jnp.full((16,), off, jnp.int32)
    for g in range(CH // 16):
        idx_v[pl.ds(g * 16, 16)] = idx_v[pl.ds(g * 16, 16)] + ov


def _sc_conv1(x_hbm, ab_hbm, src_hbm, dst_hbm, ew_hbm,
              s_out, t_out, cnt_out, deg_out,
              acc_sh, sca_sh, tA, tB, idx_v, dst_v, rows_v, q_v, w_v, sem):
    c = lax.axis_index("c")
    s = lax.axis_index("s")
    _zero_rows(rows_v)
    _zero_vec(q_v)
    pltpu.sync_copy(ab_hbm.at[0], tA)
    pltpu.sync_copy(ab_hbm.at[1], tB)
    _zero_acc(acc_sh, rows_v, s)

    @pl.when(s == 0)
    def _():
        _zero_sca(sca_sh, q_v)

    @pl.when(c == 0)
    def _():
        _ones_vec(w_v)

    plsc.subcore_barrier()

    epb = E // 16

    def chunk(i, carry):
        off = pl.multiple_of(s * epb + i * CH, CH)
        pltpu.sync_copy(src_hbm.at[pl.ds(off, CH)], idx_v)
        pltpu.sync_copy(dst_hbm.at[pl.ds(off, CH)], dst_v)
        pltpu.async_copy(x_hbm.at[idx_v], rows_v, sem).wait()

        @pl.when(c == 1)
        def _():
            _gate(idx_v, dst_v, tA, tB, q_v)
            _scale_rows(rows_v, q_v)
            pltpu.sync_copy(ew_hbm.at[pl.ds(off, CH)], w_v)

        pltpu.sync_copy(w_v, sca_sh.at[dst_v], add=True)
        pltpu.sync_copy(rows_v, acc_sh.at[dst_v], add=True)
        return carry

    lax.fori_loop(0, NCH_FULL, chunk, 0)
    plsc.subcore_barrier()

    @pl.when(c == 0)
    def _():
        _flush_acc(acc_sh, s_out, s)

        @pl.when(s == 0)
        def _():
            pltpu.sync_copy(sca_sh, cnt_out)

    @pl.when(c == 1)
    def _():
        _flush_acc(acc_sh, t_out, s)

        @pl.when(s == 0)
        def _():
            pltpu.sync_copy(sca_sh, deg_out)


def _sc_conv2(h2_hbm, ab_hbm, src_hbm, dst_hbm,
              s_out, t_out,
              acc_sh, tA, tB, idx_v, dst_v, rows_v, q_v, sem):
    c = lax.axis_index("c")
    s = lax.axis_index("s")
    pltpu.sync_copy(ab_hbm.at[0], tA)
    pltpu.sync_copy(ab_hbm.at[1], tB)
    _zero_vec(q_v)
    epb = E // 16

    for half in range(2):
        _zero_rows(rows_v)
        _zero_acc(acc_sh, rows_v, s)
        plsc.subcore_barrier()

        def chunk(i, carry):
            off = pl.multiple_of(s * epb + i * CH, CH)
            pltpu.sync_copy(src_hbm.at[pl.ds(off, CH)], idx_v)
            pltpu.sync_copy(dst_hbm.at[pl.ds(off, CH)], dst_v)

            @pl.when(c == 1)
            def _():
                _gate(idx_v, dst_v, tA, tB, q_v)

            if half:
                _offset_idx(idx_v, half * N)
            pltpu.async_copy(h2_hbm.at[idx_v], rows_v, sem).wait()

            @pl.when(c == 1)
            def _():
                _scale_rows(rows_v, q_v)

            pltpu.sync_copy(rows_v, acc_sh.at[dst_v], add=True)
            return carry

        lax.fori_loop(0, NCH_FULL, chunk, 0)
        plsc.subcore_barrier()

        @pl.when(c == 0)
        def _():
            _flush_acc(acc_sh, s_out, s, base=half * N)

        @pl.when(c == 1)
        def _():
            _flush_acc(acc_sh, t_out, s, base=half * N)


def _sc_gcn1(h3_hbm, dinv_hbm, src_hbm, dst_hbm, ew_hbm,
             q_out,
             acc_sh, dt, idx_v, dst_v, rows_v, q_v, w_v, sem):
    c = lax.axis_index("c")
    s = lax.axis_index("s")
    _zero_rows(rows_v)
    pltpu.sync_copy(dinv_hbm, dt)
    _zero_acc(acc_sh, rows_v, s)
    plsc.subcore_barrier()

    epb = E // 32
    wid = c * 16 + s

    def chunk(i, carry):
        off = pl.multiple_of(wid * epb + i * CH, CH)
        pltpu.sync_copy(src_hbm.at[pl.ds(off, CH)], idx_v)
        pltpu.sync_copy(dst_hbm.at[pl.ds(off, CH)], dst_v)
        pltpu.sync_copy(ew_hbm.at[pl.ds(off, CH)], w_v)
        pltpu.async_copy(h3_hbm.at[idx_v], rows_v, sem).wait()
        _norm(idx_v, dst_v, dt, w_v, q_v)
        _scale_rows(rows_v, q_v)
        pltpu.sync_copy(rows_v, acc_sh.at[dst_v], add=True)
        return carry

    lax.fori_loop(0, NCH_HALF, chunk, 0)
    plsc.subcore_barrier()
    _flush_acc(acc_sh, q_out, s, base=c * N)


def _sc_gcn2(e1_hbm, dinv_hbm, src_hbm, dst_hbm, ew_hbm,
             p_out,
             acc_sh, dt, idx_v, dst_v, rows_v, q_v, w_v, sem):
    c = lax.axis_index("c")
    s = lax.axis_index("s")
    _zero_rows(rows_v)
    pltpu.sync_copy(dinv_hbm, dt)
    _zero_acc(acc_sh, rows_v, s)
    plsc.subcore_barrier()

    epb = E // 16

    def chunk(i, carry):
        off = pl.multiple_of(s * epb + i * CH, CH)
        pltpu.sync_copy(src_hbm.at[pl.ds(off, CH)], idx_v)
        pltpu.sync_copy(dst_hbm.at[pl.ds(off, CH)], dst_v)
        pltpu.sync_copy(ew_hbm.at[pl.ds(off, CH)], w_v)
        _norm(idx_v, dst_v, dt, w_v, q_v)
        _offset_idx(idx_v, c * N)
        pltpu.async_copy(e1_hbm.at[idx_v], rows_v, sem).wait()
        _scale_rows(rows_v, q_v)
        pltpu.sync_copy(rows_v, acc_sh.at[dst_v], add=True)
        return carry

    lax.fori_loop(0, NCH_FULL, chunk, 0)
    plsc.subcore_barrier()
    _flush_acc(acc_sh, p_out, s, base=c * N)


def _sc_run(body, outs, scratches, args):
    mesh = plsc.VectorSubcoreMesh(core_axis_name="c", subcore_axis_name="s")
    return pl.kernel(
        body, out_type=outs, mesh=mesh, scratch_types=scratches,
        compiler_params=pltpu.CompilerParams(needs_layout_passes=False),
    )(*args)


def _edge_scratches(with_w=True, with_sca=False, table_n=2):
    sc = [pltpu.VMEM_SHARED((N, 128), f32)]
    if with_sca:
        sc.append(pltpu.VMEM_SHARED((N,), f32))
    sc += [pltpu.VMEM((N,), f32)] * table_n
    sc += [pltpu.VMEM((CH,), jnp.int32), pltpu.VMEM((CH,), jnp.int32),
           pltpu.VMEM((CH, 128), f32), pltpu.VMEM((CH,), f32)]
    if with_w:
        sc.append(pltpu.VMEM((CH,), f32))
    sc.append(pltpu.SemaphoreType.DMA)
    return sc


def _dot(a, b):
    return jnp.dot(a, b, preferred_element_type=f32)


def _tc1_body(x_ref, u_ref, c_ref, ab_ref):
    xu = _dot(x_ref[...], u_ref[...])
    d = xu[:, 0] - xu[:, 1]
    dc = c_ref[0, 0] - c_ref[0, 1]
    ab_ref[...] = jnp.stack([d + dc, d])


def _tc2_body(x_ref, s_ref, t_ref, cnt_ref, c_ref, w1_ref, b1_ref, u2_ref,
              c2_ref, h2_ref, ab2_ref):
    c0 = c_ref[0, 0]
    c1 = c_ref[0, 1]
    qs0 = 1.0 / (1.0 + jnp.exp(c1 - c0))
    x = x_ref[...]
    t0 = t_ref[...] + qs0 * x
    sm = s_ref[...] + x
    w1 = w1_ref[...]
    wd = w1[:, :HID] - w1[:, HID:]
    cnt = jnp.maximum(cnt_ref[...] + 1.0, 1.0)
    agg = (_dot(t0, wd) + _dot(sm, w1[:, HID:])) / cnt[:, None] + b1_ref[...]
    h = jnp.maximum(agg, 0.0)
    h2_ref[...] = jnp.concatenate([h[:, :128], h[:, 128:]], axis=0)
    xu = _dot(h, u2_ref[...])
    d = xu[:, 0] - xu[:, 1]
    dc2 = c2_ref[0, 0] - c2_ref[0, 1]
    ab2_ref[...] = jnp.stack([d + dc2, d])


def _tc3_body(s2_ref, t2_ref, h2_ref, cnt_ref, deg_ref, c2_ref, w2_ref,
              b2_ref, wl_ref, bl_ref, h3_ref, dinv_ref):
    c0 = c2_ref[0, 0]
    c1 = c2_ref[0, 1]
    qs2 = 1.0 / (1.0 + jnp.exp(c1 - c0))
    ha = h2_ref[:N]
    hb = h2_ref[N:]
    t2a = t2_ref[:N] + qs2 * ha
    t2b = t2_ref[N:] + qs2 * hb
    s2a = s2_ref[:N] + ha
    s2b = s2_ref[N:] + hb
    w2 = w2_ref[...]
    wd = w2[:, :LAT] - w2[:, LAT:]
    w21 = w2[:, LAT:]
    num = (_dot(t2a, wd[:128]) + _dot(t2b, wd[128:])
           + _dot(s2a, w21[:128]) + _dot(s2b, w21[128:]))
    cnt = jnp.maximum(cnt_ref[...] + 1.0, 1.0)
    lat = jnp.maximum(num / cnt[:, None] + b2_ref[...], 0.0)
    h3_ref[...] = _dot(lat, wl_ref[...]) + bl_ref[...]
    dinv_ref[...] = lax.rsqrt(deg_ref[...] + 1.0)


def _tc4_body(q_ref, h3_ref, dinv_ref, wg_ref, bg_ref, e1_ref):
    dinv = dinv_ref[...]
    qt = q_ref[:N] + q_ref[N:] + (dinv * dinv)[:, None] * h3_ref[...]
    e1 = jnp.maximum(_dot(qt, wg_ref[...]) + bg_ref[...], 0.0)
    e1_ref[...] = jnp.concatenate([e1[:, :128], e1[:, 128:]], axis=0)


def _tc5_body(p_ref, e1_ref, dinv_ref, wmu_ref, bmu_ref, wlv_ref, blv_ref,
              eps_ref, wd1_ref, bd1_ref, wd2_ref, bd2_ref,
              recon_ref, mu_ref, lv_ref):
    dinv = dinv_ref[...]
    d2 = (dinv * dinv)[:, None]
    pa = p_ref[:N] + d2 * e1_ref[:N]
    pb = p_ref[N:] + d2 * e1_ref[N:]
    wmu = wmu_ref[...]
    wlv = wlv_ref[...]
    mu = _dot(pa, wmu[:128]) + _dot(pb, wmu[128:]) + bmu_ref[...]
    lv = _dot(pa, wlv[:128]) + _dot(pb, wlv[128:]) + blv_ref[...]
    std = jnp.exp(0.5 * lv)
    z = mu + eps_ref[...] * std
    dd = jnp.maximum(_dot(z, wd1_ref[...]) + bd1_ref[...], 0.0)
    recon_ref[...] = _dot(dd, wd2_ref[...]) + bd2_ref[...]
    mu_ref[...] = mu
    lv_ref[...] = lv


def _tc_run(body, outs, args):
    return pl.pallas_call(body, out_shape=outs)(*args)


def kernel(_x, _edge_index, _edge_weight, W1, u1, c1, b1, W2, u2, c2, b2,
           Wl, bl, Wg1, bg1, Wmu, bmu, Wlv, blv, Wd1, bd1, Wd2, bd2):
    src = _edge_index[0]
    dst = _edge_index[1]
    c1r = c1.reshape(1, HEADS)
    c2r = c2.reshape(1, HEADS)
    b1r = b1.reshape(1, HID)
    b2r = b2.reshape(1, LAT)
    blr = bl.reshape(1, OUT)
    bgr = bg1.reshape(1, HID)
    bmur = bmu.reshape(1, LAT)
    blvr = blv.reshape(1, LAT)
    bd1r = bd1.reshape(1, HID)
    bd2r = bd2.reshape(1, OUT)
    eps = jax.random.normal(jax.random.key(42), (N, LAT), dtype=f32)

    sds = jax.ShapeDtypeStruct

    ab1 = _tc_run(_tc1_body, sds((2, N), f32), (_x, u1, c1r))

    s1, t1, cnt, deg = _sc_run(
        _sc_conv1,
        [sds((N, 128), f32), sds((N, 128), f32), sds((N,), f32), sds((N,), f32)],
        _edge_scratches(with_w=True, with_sca=True, table_n=2),
        (_x, ab1, src, dst, _edge_weight))

    h2cat, ab2 = _tc_run(
        _tc2_body, [sds((2 * N, 128), f32), sds((2, N), f32)],
        (_x, s1, t1, cnt, c1r, W1, b1r, u2, c2r))

    s2cat, t2cat = _sc_run(
        _sc_conv2,
        [sds((2 * N, 128), f32), sds((2 * N, 128), f32)],
        _edge_scratches(with_w=False, with_sca=False, table_n=2),
        (h2cat, ab2, src, dst))

    h3, dinv = _tc_run(
        _tc3_body, [sds((N, 128), f32), sds((N,), f32)],
        (s2cat, t2cat, h2cat, cnt, deg, c2r, W2, b2r, Wl, blr))

    qcat = _sc_run(
        _sc_gcn1, sds((2 * N, 128), f32),
        _edge_scratches(with_w=True, with_sca=False, table_n=1),
        (h3, dinv, src, dst, _edge_weight))

    e1cat = _tc_run(_tc4_body, sds((2 * N, 128), f32), (qcat, h3, dinv, Wg1, bgr))

    pcat = _sc_run(
        _sc_gcn2, sds((2 * N, 128), f32),
        _edge_scratches(with_w=True, with_sca=False, table_n=1),
        (e1cat, dinv, src, dst, _edge_weight))

    recon, mu, logvar = _tc_run(
        _tc5_body,
        [sds((N, OUT), f32), sds((N, LAT), f32), sds((N, LAT), f32)],
        (pcat, e1cat, dinv, Wmu, bmur, Wlv, blvr, eps, Wd1, bd1r, Wd2, bd2r))

    return recon, mu, logvar
